# Initial kernel scaffold; baseline (speedup 1.0000x reference)
import jax
import jax.numpy as jnp
from jax.experimental import pallas as pl


def kernel(x, patch_W, patch_b, Wl0, bl0, Wr0, Wl1, bl1, Wr1, Wl2, bl2, Wr2):
    raise NotImplementedError("write your pallas kernel here")


# speedup vs baseline: 5.8859x; 5.8859x over previous
import functools

import jax
import jax.numpy as jnp
from jax import lax
from jax.experimental import pallas as pl
from jax.experimental.pallas import tpu as pltpu
from jax.experimental.pallas import tpu_sc as plsc

N_NODES = 3136
FDIM = 128
KNN = 16
N_IMGS = 16
NODES_PER_IMG = 196
ROWS = 392
_F32 = jnp.float32
_HI = lax.Precision.HIGHEST


def _dot(a, b, dims):
    return lax.dot_general(a, b, (dims, ((), ())),
                           precision=_HI, preferred_element_type=_F32)


def _embed_body(xr_ref, w_ref, b_ref, out_ref):
    out_ref[...] = _dot(xr_ref[...], w_ref[...], ((1,), (0,))) + b_ref[...]


def _patch_embed(xr, patch_W, patch_b):
    wp = jnp.pad(patch_W, ((0, 0), (0, FDIM - patch_W.shape[1])))
    bp = jnp.pad(patch_b, (0, FDIM - patch_b.shape[0])).reshape(1, FDIM)
    return pl.pallas_call(
        _embed_body,
        out_shape=jax.ShapeDtypeStruct((N_NODES, FDIM), _F32),
    )(xr, wp, bp)


def _knn_body(fa_ref, fb_ref, idx_ref):
    fa = fa_ref[...]
    fb = fb_ref[...]
    sqa = jnp.sum(fa * fa, axis=1, keepdims=True)
    ones = jnp.ones((1, FDIM), _F32)
    sqb = _dot(ones, fb * fb, ((1,), (1,)))
    g = _dot(fa, fb, ((1,), (1,)))
    d = sqa + sqb - 2.0 * g
    col = lax.broadcasted_iota(jnp.int32, (ROWS, N_NODES), 1)
    kcol = lax.broadcasted_iota(jnp.int32, (ROWS, KNN), 1)
    acc = jnp.zeros((ROWS, KNN), jnp.int32)
    for k in range(KNN):
        m = jnp.min(d, axis=1, keepdims=True)
        j = jnp.min(jnp.where(d == m, col, jnp.int32(2 ** 30)),
                    axis=1, keepdims=True)
        acc = jnp.where(kcol == k, j, acc)
        d = jnp.where(col == j, jnp.inf, d)
    idx_ref[...] = acc


def _knn_topk(feats):
    return pl.pallas_call(
        _knn_body,
        grid=(N_NODES // ROWS,),
        in_specs=[pl.BlockSpec((ROWS, FDIM), lambda i: (i, 0)),
                  pl.BlockSpec((N_NODES, FDIM), lambda i: (0, 0))],
        out_specs=pl.BlockSpec((ROWS, KNN), lambda i: (i, 0)),
        out_shape=jax.ShapeDtypeStruct((N_NODES, KNN), jnp.int32),
    )(feats, feats)


def _gather_sum_sc(h, idx_flat):
    n, d = h.shape
    nw = 32
    npw = n // nw
    ch = 7
    nch = npw // ch

    mesh = plsc.VectorSubcoreMesh(core_axis_name="c", subcore_axis_name="s")

    @functools.partial(
        pl.kernel, mesh=mesh,
        out_type=jax.ShapeDtypeStruct((n * d,), _F32),
        scratch_types=[
            pltpu.VMEM((npw * KNN,), jnp.int32),
            pltpu.VMEM((ch * KNN, d), _F32),
            pltpu.VMEM((npw * d,), _F32),
            pltpu.SemaphoreType.DMA,
        ],
    )
    def k(h_hbm, idx_hbm, out_hbm, idx_v, rows_v, out_v, sem):
        wid = lax.axis_index("s") * 2 + lax.axis_index("c")
        pltpu.sync_copy(idx_hbm.at[pl.ds(wid * (npw * KNN), npw * KNN)], idx_v)

        def chunk(c, carry):
            pltpu.async_copy(
                h_hbm.at[idx_v.at[pl.ds(c * (ch * KNN), ch * KNN)]],
                rows_v, sem).wait()

            def node(i, carry2):
                for colb in range(d // 16):
                    s = rows_v[i * KNN, pl.ds(colb * 16, 16)]
                    for r in range(1, KNN):
                        s = s + rows_v[i * KNN + r, pl.ds(colb * 16, 16)]
                    out_v[pl.ds((c * ch + i) * d + colb * 16, 16)] = s
                return carry2

            return lax.fori_loop(0, ch, node, carry)

        lax.fori_loop(0, nch, chunk, 0)
        pltpu.sync_copy(out_v, out_hbm.at[pl.ds(wid * (npw * d), npw * d)])

    return k(h, idx_flat).reshape(n, d)


def _sage_body(sum_ref, h_ref, wl_ref, wr_ref, b_ref, out_ref):
    mean = sum_ref[...] * (1.0 / KNN)
    acc = _dot(mean, wl_ref[...], ((1,), (0,))) + b_ref[...]
    acc = acc + _dot(h_ref[...], wr_ref[...], ((1,), (0,)))
    out_ref[...] = jnp.maximum(acc, 0.0)


def _sage_layer(nb_sum, h, wl, bl, wr):
    dout = wl.shape[1]
    return pl.pallas_call(
        _sage_body,
        out_shape=jax.ShapeDtypeStruct((N_NODES, dout), _F32),
    )(nb_sum, h, wl, wr, bl.reshape(1, dout))


def _sage_pool_body(sum_ref, h_ref, wl_ref, wr_ref, b_ref, out_ref):
    mean = sum_ref[...] * (1.0 / KNN)
    acc = _dot(mean, wl_ref[...], ((1,), (0,))) + b_ref[...]
    acc = acc + _dot(h_ref[...], wr_ref[...], ((1,), (0,)))
    t = jnp.maximum(acc, 0.0)
    row = lax.broadcasted_iota(jnp.int32, (N_NODES, N_IMGS), 0)
    gcol = lax.broadcasted_iota(jnp.int32, (N_NODES, N_IMGS), 1)
    pmat = jnp.where(row // NODES_PER_IMG == gcol, 1.0, 0.0)
    out_ref[...] = _dot(pmat, t, ((0,), (0,))) * (1.0 / NODES_PER_IMG)


def _sage_pool_layer(nb_sum, h, wl, bl, wr):
    dout = wl.shape[1]
    return pl.pallas_call(
        _sage_pool_body,
        out_shape=jax.ShapeDtypeStruct((N_IMGS, dout), _F32),
    )(nb_sum, h, wl, wr, bl.reshape(1, dout))


def kernel(x, patch_W, patch_b, Wl0, bl0, Wr0, Wl1, bl1, Wr1, Wl2, bl2, Wr2):
    bn, c, hh, ww = x.shape
    p = 16
    xr = (x.reshape(bn, c, hh // p, p, ww // p, p)
           .transpose(0, 2, 4, 1, 3, 5)
           .reshape(bn * (hh // p) * (ww // p), c * p * p))
    feats = _patch_embed(xr, patch_W, patch_b)
    idx_flat = _knn_topk(feats).reshape(-1)

    wl0p = jnp.pad(Wl0, ((0, FDIM - Wl0.shape[0]), (0, 0)))
    wr0p = jnp.pad(Wr0, ((0, FDIM - Wr0.shape[0]), (0, 0)))
    s0 = _gather_sum_sc(feats, idx_flat)
    h1 = _sage_layer(s0, feats, wl0p, bl0, wr0p)
    s1 = _gather_sum_sc(h1, idx_flat)
    h2 = _sage_layer(s1, h1, Wl1, bl1, Wr1)
    s2 = _gather_sum_sc(h2, idx_flat)
    return _sage_pool_layer(s2, h2, Wl2, bl2, Wr2)
